# baseline (speedup 1.0000x reference)
import functools

import jax
import jax.numpy as jnp
from jax import lax
from jax.experimental import pallas as pl
from jax.experimental.pallas import tpu as pltpu
from jax.experimental.pallas import tpu_sc as plsc

_FM = 32
_SUB = 8


def _make_sc_gather(B):
    info = plsc.get_sparse_core_info()
    n_cores, n_sub = info.num_cores, info.num_subcores
    nw = n_cores * n_sub
    b_per_w = B // nw
    lanes = info.num_lanes

    mesh = plsc.VectorSubcoreMesh(core_axis_name="c", subcore_axis_name="s")

    @functools.partial(
        pl.kernel,
        mesh=mesh,
        out_type=[jax.ShapeDtypeStruct((B * _FM,), jnp.float32)
                  for _ in range(4)],
        scratch_types=[
            pltpu.VMEM((b_per_w,), jnp.int32),
            pltpu.VMEM((b_per_w * _FM,), jnp.float32),
            [pltpu.SemaphoreType.DMA for _ in range(4)],
        ],
    )
    def sc_gather(u_tab, i_tab, c_tab, d_tab, uid, iid, cid, did,
                  o_u, o_i, o_c, o_d, idx_v, rows_v, sems):
        wid = lax.axis_index("s") * n_cores + lax.axis_index("c")
        base = wid * b_per_w
        tabs = ((u_tab, uid, o_u), (i_tab, iid, o_i),
                (c_tab, cid, o_c), (d_tab, did, o_d))
        for t, (tab, ids, out) in enumerate(tabs):
            pltpu.sync_copy(ids.at[pl.ds(base, b_per_w)], idx_v)

            def body(c, carry, tab=tab):
                v = idx_v[pl.ds(c * lanes, lanes)] * _FM
                for j in range(lanes):
                    off = pl.multiple_of(v[j], _FM)
                    pltpu.async_copy(
                        tab.at[pl.ds(off, _FM)],
                        rows_v.at[pl.ds((c * lanes + j) * _FM, _FM)],
                        sems[j % 4])
                return carry

            lax.fori_loop(0, b_per_w // lanes, body, 0)
            qn = (b_per_w // 4) * _FM
            for q in range(4):
                pltpu.make_async_copy(
                    out.at[pl.ds(base * _FM, qn)],
                    rows_v.at[pl.ds(0, qn)], sems[q]).wait()
            pltpu.sync_copy(rows_v, out.at[pl.ds(base * _FM, b_per_w * _FM)])

    return sc_gather


def _tc_body(eu_r, ei_r, ec_r, ed_r,
             ud_r, idn_r, WuT_r, bu_r, WiT_r, bi_r, wlin_r,
             W0T_r, b0_r, W1T_r, b1_r, W2T_r, b2b_r, out_r):
    f32 = jnp.float32
    eu = eu_r[...]
    ei = ei_r[...]
    ec = ec_r[...]
    ed = ed_r[...]
    e_ud = jnp.maximum(
        jnp.dot(ud_r[...], WuT_r[...], preferred_element_type=f32) + bu_r[...], 0.0)
    e_id = jnp.maximum(
        jnp.dot(idn_r[...], WiT_r[...], preferred_element_type=f32) + bi_r[...], 0.0)

    s = eu + ei + ec + ed + e_ud + e_id
    linear_out = jnp.dot(s, wlin_r[...], preferred_element_type=f32)
    sq_of_sum = jnp.sum(s * s, axis=1, keepdims=True)
    sum_of_sq = (jnp.sum(eu * eu, axis=1, keepdims=True)
                 + jnp.sum(ei * ei, axis=1, keepdims=True)
                 + jnp.sum(ec * ec, axis=1, keepdims=True)
                 + jnp.sum(ed * ed, axis=1, keepdims=True)
                 + jnp.sum(e_ud * e_ud, axis=1, keepdims=True)
                 + jnp.sum(e_id * e_id, axis=1, keepdims=True))
    fm_out = 0.5 * (sq_of_sum - sum_of_sq)

    deep_in = jnp.concatenate([eu, ei, ec, ed, e_ud, e_id], axis=1)
    h = jnp.maximum(
        jnp.dot(deep_in, W0T_r[...], preferred_element_type=f32) + b0_r[...], 0.0)
    h = jnp.maximum(
        jnp.dot(h, W1T_r[...], preferred_element_type=f32) + b1_r[...], 0.0)
    deep_out = jnp.dot(h, W2T_r[...], preferred_element_type=f32)

    logit = linear_out + fm_out + deep_out + b2b_r[...]
    out_r[...] = 1.0 / (1.0 + jnp.exp(-logit))


def _tc_deepfm(eu, ei, ec, ed, user_dense, item_dense,
               WuT, bu2, WiT, bi2, wlin2, W0T, b02, W1T, b12, W2T, b2b,
               blk=2048):
    B = eu.shape[0]
    grid = (B // blk,)
    row = lambda i: (i, 0)
    fix = lambda i: (0, 0)
    in_specs = (
        [pl.BlockSpec((blk, _FM), row) for _ in range(4)]
        + [pl.BlockSpec((blk, user_dense.shape[1]), row),
           pl.BlockSpec((blk, item_dense.shape[1]), row)]
        + [pl.BlockSpec(w.shape, fix)
           for w in (WuT, bu2, WiT, bi2, wlin2, W0T, b02, W1T, b12, W2T, b2b)]
    )
    return pl.pallas_call(
        _tc_body,
        grid=grid,
        in_specs=in_specs,
        out_specs=pl.BlockSpec((blk, 1), row),
        out_shape=jax.ShapeDtypeStruct((B, 1), jnp.float32),
    )(eu, ei, ec, ed, user_dense, item_dense,
      WuT, bu2, WiT, bi2, wlin2, W0T, b02, W1T, b12, W2T, b2b)


def kernel(user_id, item_id, item_category, item_dur_bkt, user_dense,
           item_dense, user_tab, item_tab, cat_tab, dur_tab, Wu, bu, Wi, bi,
           w_lin, W0, b0, W1, b1, W2, b2, bias):
    B = user_id.shape[0]
    uid = user_id.astype(jnp.int32)
    iid = item_id.astype(jnp.int32)
    cid = item_category.astype(jnp.int32)
    did = item_dur_bkt.astype(jnp.int32)

    sc_gather = _make_sc_gather(B)
    eu, ei, ec, ed = sc_gather(
        user_tab.reshape(-1), item_tab.reshape(-1), cat_tab.reshape(-1),
        dur_tab.reshape(-1), uid, iid, cid, did)
    eu = eu.reshape(B, _FM)
    ei = ei.reshape(B, _FM)
    ec = ec.reshape(B, _FM)
    ed = ed.reshape(B, _FM)

    out = _tc_deepfm(
        eu, ei, ec, ed, user_dense, item_dense,
        Wu.T, bu.reshape(1, -1), Wi.T, bi.reshape(1, -1),
        w_lin.reshape(-1, 1), W0.T, b0.reshape(1, -1), W1.T,
        b1.reshape(1, -1), W2.T, (b2 + bias).reshape(1, 1))
    return out.reshape(B)


# speedup vs baseline: 1.4486x; 1.4486x over previous
import functools

import jax
import jax.numpy as jnp
from jax import lax
from jax.experimental import pallas as pl
from jax.experimental.pallas import tpu as pltpu
from jax.experimental.pallas import tpu_sc as plsc

_FM = 32
_SUB = 8


def _make_sc_gather(B):
    info = plsc.get_sparse_core_info()
    n_cores, n_sub = info.num_cores, info.num_subcores
    nw = n_cores * n_sub
    b_per_w = B // nw
    lanes = info.num_lanes

    mesh = plsc.VectorSubcoreMesh(core_axis_name="c", subcore_axis_name="s")

    @functools.partial(
        pl.kernel,
        mesh=mesh,
        out_type=[jax.ShapeDtypeStruct((B, _FM), jnp.float32)
                  for _ in range(4)],
        scratch_types=[
            pltpu.VMEM((b_per_w,), jnp.int32),
            pltpu.VMEM((b_per_w, _FM), jnp.float32),
            [pltpu.SemaphoreType.DMA for _ in range(4)],
        ],
    )
    def sc_gather(u_tab, i_tab, c_tab, d_tab, uid, iid, cid, did,
                  o_u, o_i, o_c, o_d, idx_v, rows_v, sems):
        wid = lax.axis_index("s") * n_cores + lax.axis_index("c")
        base = wid * b_per_w
        tabs = ((u_tab, uid, o_u), (i_tab, iid, o_i),
                (c_tab, cid, o_c), (d_tab, did, o_d))
        for t, (tab, ids, out) in enumerate(tabs):
            pltpu.sync_copy(ids.at[pl.ds(base, b_per_w)], idx_v)

            def body(c, carry, tab=tab):
                v = idx_v[pl.ds(c * lanes, lanes)]
                for j in range(lanes):
                    pltpu.async_copy(
                        tab.at[v[j]], rows_v.at[c * lanes + j],
                        sems[j % 4])
                return carry

            lax.fori_loop(0, b_per_w // lanes, body, 0)
            for q in range(4):
                pltpu.make_async_copy(
                    out.at[pl.ds(base, b_per_w // 4)],
                    rows_v.at[pl.ds(0, b_per_w // 4)], sems[q]).wait()
            pltpu.sync_copy(rows_v, out.at[pl.ds(base, b_per_w)])

    return sc_gather


def _tc_body(eu_r, ei_r, ec_r, ed_r,
             ud_r, idn_r, WuT_r, bu_r, WiT_r, bi_r, wlin_r,
             W0T_r, b0_r, W1T_r, b1_r, W2T_r, b2b_r, out_r):
    f32 = jnp.float32
    eu = eu_r[...]
    ei = ei_r[...]
    ec = ec_r[...]
    ed = ed_r[...]
    e_ud = jnp.maximum(
        jnp.dot(ud_r[...], WuT_r[...], preferred_element_type=f32) + bu_r[...], 0.0)
    e_id = jnp.maximum(
        jnp.dot(idn_r[...], WiT_r[...], preferred_element_type=f32) + bi_r[...], 0.0)

    s = eu + ei + ec + ed + e_ud + e_id
    linear_out = jnp.dot(s, wlin_r[...], preferred_element_type=f32)
    sq_of_sum = jnp.sum(s * s, axis=1, keepdims=True)
    sum_of_sq = (jnp.sum(eu * eu, axis=1, keepdims=True)
                 + jnp.sum(ei * ei, axis=1, keepdims=True)
                 + jnp.sum(ec * ec, axis=1, keepdims=True)
                 + jnp.sum(ed * ed, axis=1, keepdims=True)
                 + jnp.sum(e_ud * e_ud, axis=1, keepdims=True)
                 + jnp.sum(e_id * e_id, axis=1, keepdims=True))
    fm_out = 0.5 * (sq_of_sum - sum_of_sq)

    deep_in = jnp.concatenate([eu, ei, ec, ed, e_ud, e_id], axis=1)
    h = jnp.maximum(
        jnp.dot(deep_in, W0T_r[...], preferred_element_type=f32) + b0_r[...], 0.0)
    h = jnp.maximum(
        jnp.dot(h, W1T_r[...], preferred_element_type=f32) + b1_r[...], 0.0)
    deep_out = jnp.dot(h, W2T_r[...], preferred_element_type=f32)

    logit = linear_out + fm_out + deep_out + b2b_r[...]
    out_r[...] = 1.0 / (1.0 + jnp.exp(-logit))


def _tc_deepfm(eu, ei, ec, ed, user_dense, item_dense,
               WuT, bu2, WiT, bi2, wlin2, W0T, b02, W1T, b12, W2T, b2b,
               blk=2048):
    B = eu.shape[0]
    grid = (B // blk,)
    row = lambda i: (i, 0)
    fix = lambda i: (0, 0)
    in_specs = (
        [pl.BlockSpec((blk, _FM), row) for _ in range(4)]
        + [pl.BlockSpec((blk, user_dense.shape[1]), row),
           pl.BlockSpec((blk, item_dense.shape[1]), row)]
        + [pl.BlockSpec(w.shape, fix)
           for w in (WuT, bu2, WiT, bi2, wlin2, W0T, b02, W1T, b12, W2T, b2b)]
    )
    return pl.pallas_call(
        _tc_body,
        grid=grid,
        in_specs=in_specs,
        out_specs=pl.BlockSpec((blk, 1), row),
        out_shape=jax.ShapeDtypeStruct((B, 1), jnp.float32),
    )(eu, ei, ec, ed, user_dense, item_dense,
      WuT, bu2, WiT, bi2, wlin2, W0T, b02, W1T, b12, W2T, b2b)


def kernel(user_id, item_id, item_category, item_dur_bkt, user_dense,
           item_dense, user_tab, item_tab, cat_tab, dur_tab, Wu, bu, Wi, bi,
           w_lin, W0, b0, W1, b1, W2, b2, bias):
    B = user_id.shape[0]
    uid = user_id.astype(jnp.int32)
    iid = item_id.astype(jnp.int32)
    cid = item_category.astype(jnp.int32)
    did = item_dur_bkt.astype(jnp.int32)

    sc_gather = _make_sc_gather(B)
    eu, ei, ec, ed = sc_gather(
        user_tab, item_tab, cat_tab, dur_tab, uid, iid, cid, did)

    out = _tc_deepfm(
        eu, ei, ec, ed, user_dense, item_dense,
        Wu.T, bu.reshape(1, -1), Wi.T, bi.reshape(1, -1),
        w_lin.reshape(-1, 1), W0.T, b0.reshape(1, -1), W1.T,
        b1.reshape(1, -1), W2.T, (b2 + bias).reshape(1, 1))
    return out.reshape(B)
